# baseline (speedup 1.0000x reference)
import functools

import jax
import jax.numpy as jnp
from jax import lax
from jax.experimental import pallas as pl
from jax.experimental.pallas import tpu as pltpu
from jax.experimental.pallas import tpu_sc as plsc

NC = 2
NS = 16
NW = NC * NS
C = 128
L = 16


def _sc_aggregate(feature, edge_index, per, n_chunks, n_pad, D):
    mesh = plsc.VectorSubcoreMesh(
        core_axis_name="c", subcore_axis_name="s", num_cores=NC, num_subcores=NS
    )
    rows_per = n_pad // NS
    zeros = jnp.zeros((rows_per, D), jnp.float32)
    N = feature.shape[0]
    pad_rows = n_pad - N
    dmask = 1
    while dmask * 2 <= pad_rows:
        dmask *= 2

    @functools.partial(
        pl.kernel,
        mesh=mesh,
        out_type=jax.ShapeDtypeStruct((NC, n_pad, D), jnp.float32),
        scratch_types=[
            pltpu.VMEM((C,), jnp.int32),
            pltpu.VMEM((C,), jnp.int32),
            pltpu.VMEM((C,), jnp.int32),
            pltpu.VMEM((C,), jnp.int32),
            pltpu.VMEM((C, D), jnp.float32),
            pltpu.VMEM((C, D), jnp.float32),
            pltpu.VMEM_SHARED((n_pad, D), jnp.float32),
            pltpu.SemaphoreType.DMA,
            pltpu.SemaphoreType.DMA,
            pltpu.SemaphoreType.DMA,
            pltpu.SemaphoreType.DMA,
            pltpu.SemaphoreType.DMA,
            pltpu.SemaphoreType.DMA,
            pltpu.SemaphoreType.DMA,
            pltpu.SemaphoreType.DMA,
        ],
    )
    def agg_kernel(src_hbm, dst_hbm, z_hbm, feat_hbm, out_hbm,
                   s_a, d_a, s_b, d_b, rows_a, rows_b, agg_sh,
                   sem_g1, sem_g2, sem_s0, sem_s1,
                   sem_ia, sem_ib, sem_ja, sem_jb):
        c = lax.axis_index("c")
        s = lax.axis_index("s")
        wid = s * NC + c
        base = wid * per
        row0 = s * rows_per
        pltpu.sync_copy(z_hbm, agg_sh.at[pl.ds(row0, rows_per)])
        plsc.subcore_barrier()

        def stage(j, s_ref, d_ref, sem_i, sem_j):
            off = base + jnp.minimum(j * C, per - C)
            cp_s = pltpu.async_copy(src_hbm.at[pl.ds(off, C)], s_ref, sem_i)
            cp_d = pltpu.async_copy(dst_hbm.at[pl.ds(off, C)], d_ref, sem_j)
            return cp_s, cp_d

        def patch(j, s_ref, d_ref):
            nv = jnp.clip(per - j * C, 0, C)
            for g in range(C // L):
                @pl.when(g * L < C - nv)
                def _():
                    k = lax.iota(jnp.int32, L) + g * L
                    s_ref[pl.ds(g * L, L)] = k
                    d_ref[pl.ds(g * L, L)] = N + (k & (dmask - 1))

        sa0, da0 = stage(0, s_a, d_a, sem_ia, sem_ja)
        sa0.wait()
        da0.wait()
        patch(0, s_a, d_a)
        pltpu.sync_copy(feat_hbm.at[s_a], rows_a)

        @pl.loop(0, n_chunks, step=2)
        def _(j0):
            sb, db = stage(j0 + 1, s_b, d_b, sem_ib, sem_jb)
            sc0 = pltpu.async_copy(rows_a, agg_sh.at[d_a], sem_s0, add=True)
            sb.wait()
            db.wait()
            patch(j0 + 1, s_b, d_b)
            g1 = pltpu.async_copy(feat_hbm.at[s_b], rows_b, sem_g1)
            sc0.wait()
            jn = jnp.minimum(j0 + 2, n_chunks - 1)
            sa, da = stage(jn, s_a, d_a, sem_ia, sem_ja)
            g1.wait()
            sc1 = pltpu.async_copy(rows_b, agg_sh.at[d_b], sem_s1, add=True)
            sa.wait()
            da.wait()
            patch(jn, s_a, d_a)
            g2 = pltpu.async_copy(feat_hbm.at[s_a], rows_a, sem_g2)
            sc1.wait()
            g2.wait()

        plsc.subcore_barrier()
        pltpu.sync_copy(agg_sh.at[pl.ds(row0, rows_per)],
                        out_hbm.at[c, pl.ds(row0, rows_per)])

    return agg_kernel(edge_index[0], edge_index[1], zeros, feature)


def _tc_dense(feature, parts, W1, b1, W2, b2, gamma, beta, N, D):
    def body(f_ref, p_ref, w1_ref, b1_ref, w2_ref, b2_ref, g_ref, bt_ref,
             o_ref):
        h = f_ref[...] + p_ref[0, :N, :] + p_ref[1, :N, :]
        h = lax.dot_general(h, w1_ref[...], (((1,), (1,)), ((), ())),
                            preferred_element_type=jnp.float32) + b1_ref[...]
        h = jnp.maximum(h, 0.0)
        h = lax.dot_general(h, w2_ref[...], (((1,), (1,)), ((), ())),
                            preferred_element_type=jnp.float32) + b2_ref[...]
        h = jnp.maximum(h, 0.0)
        mean = jnp.mean(h, axis=0, keepdims=True)
        cent = h - mean
        var = jnp.mean(cent * cent, axis=0, keepdims=True)
        o_ref[...] = (g_ref[...] * cent * lax.rsqrt(var + 1e-5) + bt_ref[...])

    return pl.pallas_call(
        body,
        out_shape=jax.ShapeDtypeStruct((N, D), jnp.float32),
    )(feature, parts, W1, b1.reshape(1, D), W2, b2.reshape(1, D),
      gamma.reshape(1, D), beta.reshape(1, D))


def kernel(feature, edge_index, W1, b1, W2, b2, gamma, beta):
    N, D = feature.shape
    E = edge_index.shape[1]
    per = E // NW
    n_chunks = -(-per // C)
    n_chunks += n_chunks % 2
    n_pad = -(-(N + 1) // (NS * 8)) * (NS * 8)

    ei = edge_index.astype(jnp.int32)
    parts = _sc_aggregate(feature, ei, per, n_chunks, n_pad, D)
    return _tc_dense(feature, parts, W1, b1, W2, b2, gamma, beta, N, D)


# speedup vs baseline: 1.0863x; 1.0863x over previous
import functools

import jax
import jax.numpy as jnp
from jax import lax
from jax.experimental import pallas as pl
from jax.experimental.pallas import tpu as pltpu
from jax.experimental.pallas import tpu_sc as plsc

NC = 2
NS = 16
NW = NC * NS
C = 128
L = 16


def _sc_aggregate(feature, packed_w, n_chunks, n_pad, D):
    mesh = plsc.VectorSubcoreMesh(
        core_axis_name="c", subcore_axis_name="s", num_cores=NC, num_subcores=NS
    )
    rows_per = n_pad // NS
    zeros = jnp.zeros((rows_per, D), jnp.float32)

    @functools.partial(
        pl.kernel,
        mesh=mesh,
        out_type=jax.ShapeDtypeStruct((NC, n_pad, D), jnp.float32),
        scratch_types=[
            pltpu.VMEM((n_chunks, C), jnp.int32),
            pltpu.VMEM((C,), jnp.int32),
            pltpu.VMEM((C,), jnp.int32),
            pltpu.VMEM((C,), jnp.int32),
            pltpu.VMEM((C,), jnp.int32),
            pltpu.VMEM((C, D), jnp.float32),
            pltpu.VMEM((C, D), jnp.float32),
            pltpu.VMEM_SHARED((n_pad, D), jnp.float32),
            pltpu.SemaphoreType.DMA,
            pltpu.SemaphoreType.DMA,
            pltpu.SemaphoreType.DMA,
            pltpu.SemaphoreType.DMA,
        ],
    )
    def agg_kernel(packed_hbm, z_hbm, feat_hbm, out_hbm,
                   packed_v, s_a, d_a, s_b, d_b, rows_a, rows_b, agg_sh,
                   sem_g1, sem_g2, sem_s0, sem_s1):
        c = lax.axis_index("c")
        s = lax.axis_index("s")
        wid = s * NC + c
        row0 = s * rows_per
        pltpu.sync_copy(z_hbm, agg_sh.at[pl.ds(row0, rows_per)])
        pltpu.sync_copy(packed_hbm.at[wid], packed_v)
        plsc.subcore_barrier()

        def unpack(j, s_ref, d_ref):
            for g in range(C // L):
                v = packed_v[j, pl.ds(g * L, L)]
                s_ref[pl.ds(g * L, L)] = v & 0xFFFF
                d_ref[pl.ds(g * L, L)] = lax.shift_right_logical(v, 16)

        unpack(0, s_a, d_a)
        pltpu.sync_copy(feat_hbm.at[s_a], rows_a)

        @pl.loop(0, n_chunks, step=2)
        def _(j0):
            unpack(j0 + 1, s_b, d_b)
            g1 = pltpu.async_copy(feat_hbm.at[s_b], rows_b, sem_g1)
            sc0 = pltpu.async_copy(rows_a, agg_sh.at[d_a], sem_s0, add=True)
            g1.wait()
            sc1 = pltpu.async_copy(rows_b, agg_sh.at[d_b], sem_s1, add=True)
            sc0.wait()
            unpack(jnp.minimum(j0 + 2, n_chunks - 1), s_a, d_a)
            g2 = pltpu.async_copy(feat_hbm.at[s_a], rows_a, sem_g2)
            sc1.wait()
            g2.wait()

        plsc.subcore_barrier()
        pltpu.sync_copy(agg_sh.at[pl.ds(row0, rows_per)],
                        out_hbm.at[c, pl.ds(row0, rows_per)])

    return agg_kernel(packed_w, zeros, feature)


def _tc_dense(feature, parts, W1, b1, W2, b2, gamma, beta, N, D):
    def body(f_ref, p_ref, w1_ref, b1_ref, w2_ref, b2_ref, g_ref, bt_ref,
             o_ref):
        h = f_ref[...] + p_ref[0, :N, :] + p_ref[1, :N, :]
        h = lax.dot_general(h, w1_ref[...], (((1,), (1,)), ((), ())),
                            preferred_element_type=jnp.float32) + b1_ref[...]
        h = jnp.maximum(h, 0.0)
        h = lax.dot_general(h, w2_ref[...], (((1,), (1,)), ((), ())),
                            preferred_element_type=jnp.float32) + b2_ref[...]
        h = jnp.maximum(h, 0.0)
        mean = jnp.mean(h, axis=0, keepdims=True)
        cent = h - mean
        var = jnp.mean(cent * cent, axis=0, keepdims=True)
        o_ref[...] = (g_ref[...] * cent * lax.rsqrt(var + 1e-5) + bt_ref[...])

    return pl.pallas_call(
        body,
        out_shape=jax.ShapeDtypeStruct((N, D), jnp.float32),
    )(feature, parts, W1, b1.reshape(1, D), W2, b2.reshape(1, D),
      gamma.reshape(1, D), beta.reshape(1, D))


def kernel(feature, edge_index, W1, b1, W2, b2, gamma, beta):
    N, D = feature.shape
    E = edge_index.shape[1]
    per = E // NW
    n_chunks = -(-per // C)
    n_chunks += n_chunks % 2
    per_pad = n_chunks * C
    pad_cnt = per_pad - per
    n_pad = -(-(N + 1) // (NS * 8)) * (NS * 8)
    pad_rows = n_pad - N

    ei = edge_index.astype(jnp.int32)
    src = ei[0].reshape(NW, per)
    dst = ei[1].reshape(NW, per)
    pad_iota = jnp.arange(NW * pad_cnt, dtype=jnp.int32).reshape(NW, pad_cnt)
    pad_src = pad_iota % N
    pad_dst = N + pad_iota % pad_rows
    src_w = jnp.concatenate([src, pad_src], axis=1)
    dst_w = jnp.concatenate([dst, pad_dst], axis=1)
    packed_w = (src_w | (dst_w << 16)).reshape(NW, n_chunks, C)

    parts = _sc_aggregate(feature, packed_w, n_chunks, n_pad, D)
    return _tc_dense(feature, parts, W1, b1, W2, b2, gamma, beta, N, D)
